# baseline (speedup 1.0000x reference)
import functools
import jax
import jax.numpy as jnp
from jax import lax
from jax.experimental import pallas as pl
from jax.experimental.pallas import tpu as pltpu
from jax.experimental.pallas import tpu_sc as plsc

_NC, _NS, _L = 2, 16, 16
_NW = _NC * _NS
_K = 16
_PLANE = 384 * 384
_NPLANES = 64
_PPW = _NPLANES // _NW
_CHUNK = _PLANE // 4
_NCHUNK = _PPW * 4
_NBUF = 3
_UNROLL = 8


def _vgather(vec, idx):
    return lax.gather(
        vec, idx[:, None],
        dimension_numbers=lax.GatherDimensionNumbers(
            offset_dims=(), collapsed_slice_dims=(0,), start_index_map=(0,)),
        slice_sizes=(1,),
        mode=lax.GatherScatterMode.PROMISE_IN_BOUNDS)


def _sc_body(x_hbm, ev_hbm, fo_hbm, out_hbm, side_hbm,
             buf0, buf1, buf2, ev_v, fo_v, fo_ch, acc_v,
             sin0, sin1, sin2, sout0, sout1, sout2):
    bufs = (buf0, buf1, buf2)
    sins = (sin0, sin1, sin2)
    souts = (sout0, sout1, sout2)
    w = lax.axis_index("s") * _NC + lax.axis_index("c")

    pltpu.sync_copy(ev_hbm, ev_v)
    pltpu.sync_copy(fo_hbm, fo_v)

    iota = lax.iota(jnp.int32, _L)

    def chunk_off(c):
        plane = w * _PPW + c // 4
        return plane * _PLANE + (c % 4) * _CHUNK

    in_h = {}
    for c in range(min(_NBUF, _NCHUNK)):
        in_h[c] = pltpu.async_copy(
            x_hbm.at[pl.ds(chunk_off(c), _CHUNK)], bufs[c % _NBUF],
            sins[c % _NBUF])

    acc = jnp.zeros((_L,), jnp.float32)
    out_h = {}
    for c in range(_NCHUNK):
        s = c % _NBUF
        in_h[c].wait()
        out_h[c] = pltpu.async_copy(
            bufs[s], out_hbm.at[pl.ds(chunk_off(c), _CHUNK)], souts[s])

        ch = (w * _PPW + c // 4) % 4
        ev_plane = ev_v[pl.ds(ch * _K, _K)]
        fo_plane = fo_v[pl.ds(ch * _K, _K)]
        zeros = jnp.zeros((_L,), jnp.int32)
        ev0 = _vgather(ev_plane, zeros)
        ev1 = _vgather(ev_plane, zeros + 1)
        inv = 1.0 / (ev1 - ev0)
        c0 = 0.5 - ev0 * inv

        def body(i, acc):
            base = i * (_L * _UNROLL)
            for j in range(_UNROLL):
                xv = bufs[s][pl.ds(base + j * _L, _L)]
                t = xv * inv + c0
                t = jnp.minimum(jnp.maximum(t, 0.0), 15.0)
                idx = t.astype(jnp.int32)
                acc = acc + _vgather(fo_plane, idx)
            return acc

        acc = lax.fori_loop(0, _CHUNK // (_L * _UNROLL), body, acc)

        if c + _NBUF < _NCHUNK:
            out_h[c].wait()
            in_h[c + _NBUF] = pltpu.async_copy(
                x_hbm.at[pl.ds(chunk_off(c + _NBUF), _CHUNK)], bufs[s],
                sins[s])

    acc_v[...] = acc
    pltpu.sync_copy(acc_v, side_hbm.at[pl.ds(w * _L, _L)])
    for c in range(_NCHUNK - _NBUF, _NCHUNK):
        out_h[c].wait()


@functools.partial(jax.jit, static_argnums=())
def _sc_call(xf, ev2, fo2):
    n = xf.shape[0]
    mesh = plsc.VectorSubcoreMesh(
        core_axis_name="c", subcore_axis_name="s",
        num_cores=_NC, num_subcores=_NS)
    return pl.kernel(
        _sc_body,
        out_type=[
            jax.ShapeDtypeStruct((n,), jnp.float32),
            jax.ShapeDtypeStruct((_NW * _L,), jnp.float32),
        ],
        mesh=mesh,
        scratch_types=[
            pltpu.VMEM((_CHUNK,), jnp.float32),
            pltpu.VMEM((_CHUNK,), jnp.float32),
            pltpu.VMEM((_CHUNK,), jnp.float32),
            pltpu.VMEM((4 * _K,), jnp.float32),
            pltpu.VMEM((4 * _K,), jnp.float32),
            pltpu.VMEM((_K,), jnp.float32),
            pltpu.VMEM((_L,), jnp.float32),
            pltpu.SemaphoreType.DMA,
            pltpu.SemaphoreType.DMA,
            pltpu.SemaphoreType.DMA,
            pltpu.SemaphoreType.DMA,
            pltpu.SemaphoreType.DMA,
            pltpu.SemaphoreType.DMA,
        ],
    )(xf, ev2, fo2)


def kernel(x, evaluate_tables, focus_tables):
    B, C, H, W = x.shape
    xf = x.reshape(-1)
    ev2 = evaluate_tables.reshape(-1)
    fo2 = focus_tables.reshape(-1)
    out_flat, _ = _sc_call(xf, ev2, fo2)
    return out_flat.reshape(B, C, H, W)


# speedup vs baseline: 3.8026x; 3.8026x over previous
import jax
import jax.numpy as jnp
from jax.experimental import pallas as pl
from jax.experimental.pallas import tpu as pltpu


def _body(ev_ref, fo_ref, x_ref, out_ref, acc_ref, sem):
    cp = pltpu.make_async_copy(x_ref, out_ref, sem)
    cp.start()

    k_max = jnp.float32(15.0)
    for c in range(x_ref.shape[1]):
        x = x_ref[0, c]

        ev0 = ev_ref[c, 0, 0]
        inv = 1.0 / (ev_ref[c, 1, 0] - ev0)
        c0 = 0.5 - ev0 * inv
        idx_f = jnp.clip(jnp.floor(x * inv + c0), 0.0, k_max)

        f0 = fo_ref[c, 0, 0]
        fstep = fo_ref[c, 1, 0] - f0
        acc_ref[0, c, 0, 0] = (f0 * jnp.float32(x.size)
                               + fstep * jnp.sum(idx_f))

    cp.wait()


def kernel(x, evaluate_tables, focus_tables):
    B, C, H, W = x.shape
    out, _ = pl.pallas_call(
        _body,
        grid=(B,),
        in_specs=[
            pl.BlockSpec(memory_space=pltpu.SMEM),
            pl.BlockSpec(memory_space=pltpu.SMEM),
            pl.BlockSpec((1, C, H, W), lambda b: (b, 0, 0, 0)),
        ],
        out_specs=[
            pl.BlockSpec((1, C, H, W), lambda b: (b, 0, 0, 0)),
            pl.BlockSpec((1, C, 1, 1), lambda b: (b, 0, 0, 0),
                         memory_space=pltpu.SMEM),
        ],
        out_shape=[
            jax.ShapeDtypeStruct((B, C, H, W), x.dtype),
            jax.ShapeDtypeStruct((B, C, 1, 1), jnp.float32),
        ],
        scratch_shapes=[pltpu.SemaphoreType.DMA],
        compiler_params=pltpu.CompilerParams(
            dimension_semantics=("parallel",),
        ),
    )(evaluate_tables, focus_tables, x)
    return out


# speedup vs baseline: 4.4756x; 1.1770x over previous
import jax
import jax.numpy as jnp
from jax.experimental import pallas as pl
from jax.experimental.pallas import tpu as pltpu

_RB = 2


def _body(ev_ref, fo_ref, x_ref, out_ref, acc_ref):
    k_max = jnp.float32(15.0)
    for b in range(_RB):
        for c in range(x_ref.shape[1]):
            x = x_ref[b, c]

            ev0 = ev_ref[c, 0, 0]
            inv = 1.0 / (ev_ref[c, 1, 0] - ev0)
            c0 = 0.5 - ev0 * inv
            idx_f = jnp.clip(jnp.floor(x * inv + c0), 0.0, k_max)

            f0 = fo_ref[c, 0, 0]
            fstep = fo_ref[c, 1, 0] - f0
            acc_ref[b, c, 0, 0] = (f0 * jnp.float32(x.size)
                                   + fstep * jnp.sum(idx_f))

            out_ref[b, c] = x


def kernel(x, evaluate_tables, focus_tables):
    B, C, H, W = x.shape
    out, _ = pl.pallas_call(
        _body,
        grid=(B // _RB,),
        in_specs=[
            pl.BlockSpec(memory_space=pltpu.SMEM),
            pl.BlockSpec(memory_space=pltpu.SMEM),
            pl.BlockSpec((_RB, C, H, W), lambda b: (b, 0, 0, 0)),
        ],
        out_specs=[
            pl.BlockSpec((_RB, C, H, W), lambda b: (b, 0, 0, 0)),
            pl.BlockSpec((_RB, C, 1, 1), lambda b: (b, 0, 0, 0),
                         memory_space=pltpu.SMEM),
        ],
        out_shape=[
            jax.ShapeDtypeStruct((B, C, H, W), x.dtype),
            jax.ShapeDtypeStruct((B, C, 1, 1), jnp.float32),
        ],
        compiler_params=pltpu.CompilerParams(
            dimension_semantics=("parallel",),
        ),
    )(evaluate_tables, focus_tables, x)
    return out
